# baseline (speedup 1.0000x reference)
import functools
import math

import jax
import jax.numpy as jnp
from jax import lax
from jax.experimental import pallas as pl
from jax.experimental.pallas import tpu as pltpu
from jax.experimental.pallas import tpu_sc as plsc

D_MODEL = 64
SCALE = math.sqrt(D_MODEL)
NC, NS, L = 2, 16, 16
NW = NC * NS
BT = 128


def _make_sc_lookup(S, T, V, D):
    assert S == NW * BT and D == D_MODEL and V % 2 == 0 and T % 2 == 0
    mesh = plsc.VectorSubcoreMesh(core_axis_name="c", subcore_axis_name="s")

    @functools.partial(
        pl.kernel,
        mesh=mesh,
        out_type=jax.ShapeDtypeStruct((T, D // 8, NW, 8, BT), jnp.float32),
        scratch_types=[
            pltpu.VMEM((T, BT), jnp.int32),
            pltpu.VMEM((T, BT), jnp.int32),
            pltpu.VMEM((BT, 128), jnp.float32),
            pltpu.VMEM((BT, 128), jnp.float32),
            pltpu.VMEM((D // 8, 8, BT), jnp.float32),
            pltpu.SemaphoreType.DMA,
            pltpu.SemaphoreType.DMA,
        ],
        compiler_params=pltpu.CompilerParams(
            use_tc_tiling_on_sc=True, needs_layout_passes=False
        ),
    )
    def lookup(xt_hbm, t2_hbm, out_hbm, idx_v, pv, buf0, buf1, obuf, sem0, sem1):
        wid = lax.axis_index("s") * NC + lax.axis_index("c")
        pltpu.sync_copy(xt_hbm.at[:, pl.ds(wid * BT, BT)], idx_v)

        def prep(r, c):
            for k in range(BT // L):
                sl = pl.ds(k * L, L)
                v = idx_v[r, sl]
                pv[r, sl] = lax.shift_left(lax.bitwise_and(v, 1), 6)
                idx_v[r, sl] = lax.shift_right_logical(v, 1)
            return c

        lax.fori_loop(0, T, prep, 0)

        bufs = (buf0, buf1)
        sems = (sem0, sem1)

        def start_gather(t, b):
            pltpu.async_copy(t2_hbm.at[idx_v.at[t]], bufs[b], sems[b])

        def wait_gather(t, b):
            pltpu.make_async_copy(
                t2_hbm.at[idx_v.at[t]], bufs[b], sems[b]
            ).wait()

        def unit(t, b):
            wait_gather(t, b)
            buf = bufs[b]

            def kgroup(k, c):
                rows = lax.broadcasted_iota(jnp.int32, (L,), 0) + k * L
                hk = pv[t, pl.ds(k * L, L)]
                for d in range(D):
                    v = plsc.load_gather(buf, [rows, hk + d]) * SCALE
                    obuf[d // 8, d % 8, pl.ds(k * L, L)] = v
                return c

            lax.fori_loop(0, BT // L, kgroup, 0)
            pltpu.sync_copy(obuf, out_hbm.at[t, :, wid])

        start_gather(0, 0)

        def pair(p, c):
            t0 = 2 * p
            start_gather(t0 + 1, 1)
            unit(t0, 0)

            @pl.when(t0 + 2 < T)
            def _():
                start_gather(t0 + 2, 0)

            unit(t0 + 1, 1)
            return c

        lax.fori_loop(0, T // 2, pair, 0)

    return lookup


def kernel(x, table):
    S, T = x.shape
    V, D = table.shape
    xt = x.T
    t2 = table.reshape(V // 2, 2 * D)
    out5 = _make_sc_lookup(S, T, V, D)(xt, t2)
    return out5.transpose(2, 4, 0, 1, 3).reshape(S, T, D)


# speedup vs baseline: 1.0344x; 1.0344x over previous
import functools
import math

import jax
import jax.numpy as jnp
from jax import lax
from jax.experimental import pallas as pl
from jax.experimental.pallas import tpu as pltpu
from jax.experimental.pallas import tpu_sc as plsc

D_MODEL = 64
SCALE = math.sqrt(D_MODEL)
NC, NS, L = 2, 16, 16
NW = NC * NS
BT = 128


def _make_sc_lookup(S, T, V, D):
    assert S == NW * BT and D == D_MODEL and V % 2 == 0 and T % 2 == 0
    mesh = plsc.VectorSubcoreMesh(core_axis_name="c", subcore_axis_name="s")

    @functools.partial(
        pl.kernel,
        mesh=mesh,
        out_type=jax.ShapeDtypeStruct((T, D // 8, NW, 8, BT), jnp.float32),
        scratch_types=[
            pltpu.VMEM((T, BT), jnp.int32),
            pltpu.VMEM((T, BT), jnp.int32),
            pltpu.VMEM((BT, 128), jnp.float32),
            pltpu.VMEM((BT, 128), jnp.float32),
            pltpu.VMEM((D // 8, 8, BT), jnp.float32),
            pltpu.VMEM((D // 8, 8, BT), jnp.float32),
            pltpu.SemaphoreType.DMA,
            pltpu.SemaphoreType.DMA,
            pltpu.SemaphoreType.DMA,
            pltpu.SemaphoreType.DMA,
        ],
        compiler_params=pltpu.CompilerParams(
            use_tc_tiling_on_sc=True, needs_layout_passes=False
        ),
    )
    def lookup(
        xt_hbm,
        t2_hbm,
        out_hbm,
        idx_v,
        pv,
        buf0,
        buf1,
        obuf0,
        obuf1,
        sem0,
        sem1,
        osem0,
        osem1,
    ):
        wid = lax.axis_index("s") * NC + lax.axis_index("c")
        pltpu.sync_copy(xt_hbm.at[:, pl.ds(wid * BT, BT)], idx_v)

        def prep(r, c):
            for k in range(BT // L):
                sl = pl.ds(k * L, L)
                v = idx_v[r, sl]
                pv[r, sl] = lax.shift_left(lax.bitwise_and(v, 1), 6)
                idx_v[r, sl] = lax.shift_right_logical(v, 1)
            return c

        lax.fori_loop(0, T, prep, 0)

        bufs = (buf0, buf1)
        sems = (sem0, sem1)
        obufs = (obuf0, obuf1)
        osems = (osem0, osem1)

        def start_gather(t, b):
            pltpu.async_copy(t2_hbm.at[idx_v.at[t]], bufs[b], sems[b])

        def wait_gather(t, b):
            pltpu.make_async_copy(
                t2_hbm.at[idx_v.at[t]], bufs[b], sems[b]
            ).wait()

        def start_out(t, b):
            pltpu.async_copy(obufs[b], out_hbm.at[t, :, wid], osems[b])

        def wait_out(t, b):
            pltpu.make_async_copy(
                obufs[b], out_hbm.at[t, :, wid], osems[b]
            ).wait()

        def unit(t, b, first):
            if not first:
                wait_out(t - 2, b)
            wait_gather(t, b)
            buf = bufs[b]
            obuf = obufs[b]

            def kgroup(k, c):
                rows = lax.broadcasted_iota(jnp.int32, (L,), 0) + k * L
                hk = pv[t, pl.ds(k * L, L)]
                for d in range(D):
                    v = plsc.load_gather(buf, [rows, hk + d]) * SCALE
                    obuf[d // 8, d % 8, pl.ds(k * L, L)] = v
                return c

            lax.fori_loop(0, BT // L, kgroup, 0)
            start_out(t, b)

        start_gather(0, 0)
        start_gather(1, 1)
        unit(0, 0, True)
        start_gather(2, 0)
        unit(1, 1, True)

        def pair(p, c):
            t0 = 2 * p

            @pl.when(t0 + 3 < T)
            def _():
                start_gather(t0 + 3, 1)

            unit(t0 + 2, 0, False)

            @pl.when(t0 + 4 < T)
            def _():
                start_gather(t0 + 4, 0)

            unit(t0 + 3, 1, False)
            return c

        lax.fori_loop(0, (T - 2) // 2, pair, 0)
        wait_out(T - 2, 0)
        wait_out(T - 1, 1)

    return lookup


def kernel(x, table):
    S, T = x.shape
    V, D = table.shape
    xt = x.T
    t2 = table.reshape(V // 2, 2 * D)
    out5 = _make_sc_lookup(S, T, V, D)(xt, t2)
    return out5.transpose(2, 4, 0, 1, 3).reshape(S, T, D)


# speedup vs baseline: 1.5172x; 1.4667x over previous
import functools
import math

import jax
import jax.numpy as jnp
from jax import lax
from jax.experimental import pallas as pl
from jax.experimental.pallas import tpu as pltpu
from jax.experimental.pallas import tpu_sc as plsc

D_MODEL = 64
SCALE = math.sqrt(D_MODEL)
NC, NS, L = 2, 16, 16
NW = NC * NS
BT = 128
CBLK = 4096


def _make_stage1(V, D):
    nsteps = (V + CBLK - 1) // CBLK

    def body(tt_ref, out_ref):
        out_ref[:, 0:D] = tt_ref[...].T * SCALE

    return pl.pallas_call(
        body,
        grid=(nsteps,),
        in_specs=[pl.BlockSpec((D, CBLK), lambda c: (0, c))],
        out_specs=pl.BlockSpec((CBLK, 2 * D), lambda c: (c, 0)),
        out_shape=jax.ShapeDtypeStruct((V, 2 * D), jnp.float32),
    )


def _make_stage2(S, T, V, D):
    assert S == NW * BT and D == D_MODEL
    mesh = plsc.VectorSubcoreMesh(core_axis_name="c", subcore_axis_name="s")

    @functools.partial(
        pl.kernel,
        mesh=mesh,
        out_type=jax.ShapeDtypeStruct((T, D // 8, NW, 8, BT), jnp.float32),
        scratch_types=[
            pltpu.VMEM((T, BT), jnp.int32),
            pltpu.VMEM((BT, 2 * D), jnp.float32),
            pltpu.VMEM((BT, 2 * D), jnp.float32),
            pltpu.VMEM((D // 8, 8, BT), jnp.float32),
            pltpu.VMEM((D // 8, 8, BT), jnp.float32),
            pltpu.SemaphoreType.DMA,
            pltpu.SemaphoreType.DMA,
            pltpu.SemaphoreType.DMA,
            pltpu.SemaphoreType.DMA,
        ],
        compiler_params=pltpu.CompilerParams(
            use_tc_tiling_on_sc=True, needs_layout_passes=False
        ),
    )
    def lookup(
        xt_hbm,
        t2_hbm,
        out_hbm,
        idx_v,
        buf0,
        buf1,
        obuf0,
        obuf1,
        sem0,
        sem1,
        osem0,
        osem1,
    ):
        wid = lax.axis_index("s") * NC + lax.axis_index("c")
        pltpu.sync_copy(xt_hbm.at[:, pl.ds(wid * BT, BT)], idx_v)

        bufs = (buf0, buf1)
        sems = (sem0, sem1)
        obufs = (obuf0, obuf1)
        osems = (osem0, osem1)

        iota = lax.broadcasted_iota(jnp.int32, (L,), 0)
        dhis = tuple(
            lax.shift_right_logical(iota + m * L, 3) for m in range(D // L)
        )
        dlos = tuple(lax.bitwise_and(iota + m * L, 7) for m in range(D // L))

        def start_gather(t, b):
            pltpu.async_copy(t2_hbm.at[idx_v.at[t]], bufs[b], sems[b])

        def wait_gather(t, b):
            pltpu.make_async_copy(
                t2_hbm.at[idx_v.at[t]], bufs[b], sems[b]
            ).wait()

        def start_out(t, b):
            pltpu.async_copy(obufs[b], out_hbm.at[t, :, wid], osems[b])

        def wait_out(t, b):
            pltpu.make_async_copy(
                obufs[b], out_hbm.at[t, :, wid], osems[b]
            ).wait()

        def unit(t, b, first):
            if not first:
                wait_out(t - 2, b)
            wait_gather(t, b)
            buf = bufs[b]
            obuf = obufs[b]

            def blgroup(bl, c):
                dh, dl = c
                blv = jnp.full((L,), 0, jnp.int32) + bl
                for m in range(D // L):
                    v = buf[bl, pl.ds(m * L, L)]
                    plsc.store_scatter(obuf, [dh[m], dl[m], blv], v)
                return c

            lax.fori_loop(0, BT, blgroup, (dhis, dlos))
            start_out(t, b)

        start_gather(0, 0)
        start_gather(1, 1)
        unit(0, 0, True)
        start_gather(2, 0)
        unit(1, 1, True)

        def pair(p, c):
            t0 = 2 * p

            @pl.when(t0 + 3 < T)
            def _():
                start_gather(t0 + 3, 1)

            unit(t0 + 2, 0, False)

            @pl.when(t0 + 4 < T)
            def _():
                start_gather(t0 + 4, 0)

            unit(t0 + 3, 1, False)
            return c

        lax.fori_loop(0, (T - 2) // 2, pair, 0)
        wait_out(T - 2, 0)
        wait_out(T - 1, 1)

    return lookup


def kernel(x, table):
    S, T = x.shape
    V, D = table.shape
    xt = x.T
    t2 = _make_stage1(V, D)(table.T)
    out5 = _make_stage2(S, T, V, D)(xt, t2)
    return out5.transpose(2, 4, 0, 1, 3).reshape(S, T, D)


# speedup vs baseline: 1.5334x; 1.0107x over previous
import functools
import math

import jax
import jax.numpy as jnp
from jax import lax
from jax.experimental import pallas as pl
from jax.experimental.pallas import tpu as pltpu
from jax.experimental.pallas import tpu_sc as plsc

D_MODEL = 64
SCALE = math.sqrt(D_MODEL)
NC, NS, L = 2, 16, 16
NW = NC * NS
BT = 128
CBLK = 4096


def _make_stage1(V, D):
    nsteps = (V + CBLK - 1) // CBLK

    def body(tt_ref, out_ref):
        out_ref[:, 0:D] = tt_ref[...].T * SCALE

    return pl.pallas_call(
        body,
        grid=(nsteps,),
        in_specs=[pl.BlockSpec((D, CBLK), lambda c: (0, c))],
        out_specs=pl.BlockSpec((CBLK, 2 * D), lambda c: (c, 0)),
        out_shape=jax.ShapeDtypeStruct((V, 2 * D), jnp.float32),
    )


def _make_stage2(S, T, V, D):
    assert S == NW * BT and D == D_MODEL
    mesh = plsc.VectorSubcoreMesh(core_axis_name="c", subcore_axis_name="s")

    @functools.partial(
        pl.kernel,
        mesh=mesh,
        out_type=jax.ShapeDtypeStruct((T, D // 8, NW, 8, BT), jnp.float32),
        scratch_types=[
            pltpu.VMEM((T, BT), jnp.int32),
            pltpu.VMEM((BT, 2 * D), jnp.float32),
            pltpu.VMEM((BT, 2 * D), jnp.float32),
            pltpu.VMEM((D // 8, 8, BT), jnp.float32),
            pltpu.VMEM((D // 8, 8, BT), jnp.float32),
            pltpu.SemaphoreType.DMA,
            pltpu.SemaphoreType.DMA,
            pltpu.SemaphoreType.DMA,
            pltpu.SemaphoreType.DMA,
        ],
        compiler_params=pltpu.CompilerParams(
            use_tc_tiling_on_sc=True, needs_layout_passes=False
        ),
    )
    def lookup(
        xt_hbm,
        t2_hbm,
        out_hbm,
        idx_v,
        buf0,
        buf1,
        obuf0,
        obuf1,
        sem0,
        sem1,
        osem0,
        osem1,
    ):
        wid = lax.axis_index("s") * NC + lax.axis_index("c")
        pltpu.sync_copy(xt_hbm.at[:, pl.ds(wid * BT, BT)], idx_v)

        bufs = (buf0, buf1)
        sems = (sem0, sem1)
        obufs = (obuf0, obuf1)
        osems = (osem0, osem1)

        iota = lax.broadcasted_iota(jnp.int32, (L,), 0)
        dhis = tuple(
            lax.shift_right_logical(iota + m * L, 3) for m in range(D // L)
        )
        dlos = tuple(lax.bitwise_and(iota + m * L, 7) for m in range(D // L))

        def start_gather(t, b):
            pltpu.async_copy(t2_hbm.at[idx_v.at[t]], bufs[b], sems[b])

        def wait_gather(t, b):
            pltpu.make_async_copy(
                t2_hbm.at[idx_v.at[t]], bufs[b], sems[b]
            ).wait()

        def start_out(t, b):
            pltpu.async_copy(obufs[b], out_hbm.at[t, :, wid], osems[b])

        def wait_out(t, b):
            pltpu.make_async_copy(
                obufs[b], out_hbm.at[t, :, wid], osems[b]
            ).wait()

        def unit(t, b, first):
            if not first:
                wait_out(t - 2, b)
            wait_gather(t, b)
            buf = bufs[b]
            obuf = obufs[b]

            def blgroup(g, c):
                dh, dl = c
                vs = []
                for u in range(2):
                    bl = 2 * g + u
                    for m in range(D // L):
                        vs.append((bl, m, buf[bl, pl.ds(m * L, L)]))
                for bl, m, v in vs:
                    blv = jnp.full((L,), 0, jnp.int32) + bl
                    plsc.store_scatter(obuf, [dh[m], dl[m], blv], v)
                return c

            lax.fori_loop(0, BT // 2, blgroup, (dhis, dlos))
            start_out(t, b)

        start_gather(0, 0)
        start_gather(1, 1)
        unit(0, 0, True)
        start_gather(2, 0)
        unit(1, 1, True)

        def pair(p, c):
            t0 = 2 * p

            @pl.when(t0 + 3 < T)
            def _():
                start_gather(t0 + 3, 1)

            unit(t0 + 2, 0, False)

            @pl.when(t0 + 4 < T)
            def _():
                start_gather(t0 + 4, 0)

            unit(t0 + 3, 1, False)
            return c

        lax.fori_loop(0, (T - 2) // 2, pair, 0)
        wait_out(T - 2, 0)
        wait_out(T - 1, 1)

    return lookup


def kernel(x, table):
    S, T = x.shape
    V, D = table.shape
    xt = x.T
    t2 = _make_stage1(V, D)(table.T)
    out5 = _make_stage2(S, T, V, D)(xt, t2)
    return out5.transpose(2, 4, 0, 1, 3).reshape(S, T, D)


# speedup vs baseline: 1.5342x; 1.0005x over previous
import functools
import math

import jax
import jax.numpy as jnp
from jax import lax
from jax.experimental import pallas as pl
from jax.experimental.pallas import tpu as pltpu
from jax.experimental.pallas import tpu_sc as plsc

D_MODEL = 64
SCALE = math.sqrt(D_MODEL)
NC, NS, L = 2, 16, 16
NW = NC * NS
BT = 128
CBLK = 4096


def _make_stage1(V, D):
    nsteps = (V + CBLK - 1) // CBLK

    def body(tt_ref, out_ref):
        out_ref[:, 0:D] = tt_ref[...].T * SCALE

    return pl.pallas_call(
        body,
        grid=(nsteps,),
        in_specs=[pl.BlockSpec((D, CBLK), lambda c: (0, c))],
        out_specs=pl.BlockSpec((CBLK, 2 * D), lambda c: (c, 0)),
        out_shape=jax.ShapeDtypeStruct((V, 2 * D), jnp.float32),
    )


def _make_stage2(S, T, V, D):
    assert S == NW * BT and D == D_MODEL
    mesh = plsc.VectorSubcoreMesh(core_axis_name="c", subcore_axis_name="s")

    @functools.partial(
        pl.kernel,
        mesh=mesh,
        out_type=jax.ShapeDtypeStruct((T, D // 8, NW, 8, BT), jnp.float32),
        scratch_types=[
            pltpu.VMEM((T, BT), jnp.int32),
            pltpu.VMEM((BT, 2 * D), jnp.float32),
            pltpu.VMEM((BT, 2 * D), jnp.float32),
            pltpu.VMEM((D // 8, 8, BT + 1), jnp.float32),
            pltpu.VMEM((D // 8, 8, BT + 1), jnp.float32),
            pltpu.SemaphoreType.DMA,
            pltpu.SemaphoreType.DMA,
            pltpu.SemaphoreType.DMA,
            pltpu.SemaphoreType.DMA,
        ],
        compiler_params=pltpu.CompilerParams(
            use_tc_tiling_on_sc=True, needs_layout_passes=False
        ),
    )
    def lookup(
        xt_hbm,
        t2_hbm,
        out_hbm,
        idx_v,
        buf0,
        buf1,
        obuf0,
        obuf1,
        sem0,
        sem1,
        osem0,
        osem1,
    ):
        wid = lax.axis_index("s") * NC + lax.axis_index("c")
        pltpu.sync_copy(xt_hbm.at[:, pl.ds(wid * BT, BT)], idx_v)

        bufs = (buf0, buf1)
        sems = (sem0, sem1)
        obufs = (obuf0, obuf1)
        osems = (osem0, osem1)

        iota = lax.broadcasted_iota(jnp.int32, (L,), 0)
        dhis = tuple(
            lax.shift_right_logical(iota + m * L, 3) for m in range(D // L)
        )
        dlos = tuple(lax.bitwise_and(iota + m * L, 7) for m in range(D // L))

        def start_gather(t, b):
            pltpu.async_copy(t2_hbm.at[idx_v.at[t]], bufs[b], sems[b])

        def wait_gather(t, b):
            pltpu.make_async_copy(
                t2_hbm.at[idx_v.at[t]], bufs[b], sems[b]
            ).wait()

        def start_out(t, b):
            pltpu.async_copy(
                obufs[b].at[:, :, pl.ds(0, BT)], out_hbm.at[t, :, wid], osems[b]
            )

        def wait_out(t, b):
            pltpu.make_async_copy(
                obufs[b].at[:, :, pl.ds(0, BT)], out_hbm.at[t, :, wid], osems[b]
            ).wait()

        def unit(t, b, first):
            if not first:
                wait_out(t - 2, b)
            wait_gather(t, b)
            buf = bufs[b]
            obuf = obufs[b]

            def blgroup(g, c):
                dh, dl = c
                vs = []
                for u in range(2):
                    bl = 2 * g + u
                    for m in range(D // L):
                        vs.append((bl, m, buf[bl, pl.ds(m * L, L)]))
                for bl, m, v in vs:
                    blv = jnp.full((L,), 0, jnp.int32) + bl
                    plsc.store_scatter(obuf, [dh[m], dl[m], blv], v)
                return c

            lax.fori_loop(0, BT // 2, blgroup, (dhis, dlos))
            start_out(t, b)

        start_gather(0, 0)
        start_gather(1, 1)
        unit(0, 0, True)
        start_gather(2, 0)
        unit(1, 1, True)

        def pair(p, c):
            t0 = 2 * p

            @pl.when(t0 + 3 < T)
            def _():
                start_gather(t0 + 3, 1)

            unit(t0 + 2, 0, False)

            @pl.when(t0 + 4 < T)
            def _():
                start_gather(t0 + 4, 0)

            unit(t0 + 3, 1, False)
            return c

        lax.fori_loop(0, (T - 2) // 2, pair, 0)
        wait_out(T - 2, 0)
        wait_out(T - 1, 1)

    return lookup


def kernel(x, table):
    S, T = x.shape
    V, D = table.shape
    xt = x.T
    t2 = _make_stage1(V, D)(table.T)
    out5 = _make_stage2(S, T, V, D)(xt, t2)
    return out5.transpose(2, 4, 0, 1, 3).reshape(S, T, D)


# speedup vs baseline: 2.0330x; 1.3252x over previous
import functools
import math

import jax
import jax.numpy as jnp
from jax import lax
from jax.experimental import pallas as pl
from jax.experimental.pallas import tpu as pltpu
from jax.experimental.pallas import tpu_sc as plsc

D_MODEL = 64
SCALE = math.sqrt(D_MODEL)
NC, NS, L = 2, 16, 16
NW = NC * NS
BT = 128
CBLK = 8192


def _make_stage1(V, D):
    nsteps = (V + CBLK - 1) // CBLK

    def body(tt_ref, out_ref):
        out_ref[:, 0:D] = tt_ref[...].T * SCALE

    return pl.pallas_call(
        body,
        grid=(nsteps,),
        in_specs=[pl.BlockSpec((D, CBLK), lambda c: (0, c))],
        out_specs=pl.BlockSpec((CBLK, 2 * D), lambda c: (c, 0)),
        out_shape=jax.ShapeDtypeStruct((V, 2 * D), jnp.float32),
    )


def _make_stage2(S, T, V, D):
    assert S == NW * BT and D == D_MODEL
    mesh = plsc.VectorSubcoreMesh(core_axis_name="c", subcore_axis_name="s")

    @functools.partial(
        pl.kernel,
        mesh=mesh,
        out_type=jax.ShapeDtypeStruct((T, D // 8, NW, 8, BT), jnp.float32),
        scratch_types=[
            pltpu.VMEM((T, BT), jnp.int32),
            pltpu.VMEM((BT, 2 * D), jnp.float32),
            pltpu.VMEM((BT, 2 * D), jnp.float32),
            pltpu.VMEM((D // 8, 8, BT + 1), jnp.float32),
            pltpu.VMEM((D // 8, 8, BT + 1), jnp.float32),
            pltpu.SemaphoreType.DMA,
            pltpu.SemaphoreType.DMA,
            pltpu.SemaphoreType.DMA,
            pltpu.SemaphoreType.DMA,
        ],
        compiler_params=pltpu.CompilerParams(
            use_tc_tiling_on_sc=True, needs_layout_passes=False
        ),
    )
    def lookup(
        xt_hbm,
        t2_hbm,
        out_hbm,
        idx_v,
        buf0,
        buf1,
        obuf0,
        obuf1,
        sem0,
        sem1,
        osem0,
        osem1,
    ):
        wid = lax.axis_index("s") * NC + lax.axis_index("c")
        pltpu.sync_copy(xt_hbm.at[:, pl.ds(wid * BT, BT)], idx_v)

        bufs = (buf0, buf1)
        sems = (sem0, sem1)
        obufs = (obuf0, obuf1)
        osems = (osem0, osem1)

        iota = lax.broadcasted_iota(jnp.int32, (L,), 0)

        def start_gather(t, b):
            pltpu.async_copy(t2_hbm.at[idx_v.at[t]], bufs[b], sems[b])

        def wait_gather(t, b):
            pltpu.make_async_copy(
                t2_hbm.at[idx_v.at[t]], bufs[b], sems[b]
            ).wait()

        def start_out(t, b):
            pltpu.async_copy(
                obufs[b].at[:, :, pl.ds(0, BT)], out_hbm.at[t, :, wid], osems[b]
            )

        def wait_out(t, b):
            pltpu.make_async_copy(
                obufs[b].at[:, :, pl.ds(0, BT)], out_hbm.at[t, :, wid], osems[b]
            ).wait()

        def unit(t, b, first):
            if not first:
                wait_out(t - 2, b)
            wait_gather(t, b)
            buf = bufs[b]
            obuf = obufs[b]

            def kgroup(k, c):
                blv = iota + k * L
                for dg in range(D // 8):
                    vs = [
                        plsc.load_gather(
                            buf, [blv, jnp.full((L,), dg * 8 + j, jnp.int32)]
                        )
                        for j in range(8)
                    ]
                    for j, v in enumerate(vs):
                        d = dg * 8 + j
                        obuf[d // 8, d % 8, pl.ds(k * L, L)] = v
                return c

            lax.fori_loop(0, BT // L, kgroup, 0)
            start_out(t, b)

        start_gather(0, 0)
        start_gather(1, 1)
        unit(0, 0, True)
        start_gather(2, 0)
        unit(1, 1, True)

        def pair(p, c):
            t0 = 2 * p

            @pl.when(t0 + 3 < T)
            def _():
                start_gather(t0 + 3, 1)

            unit(t0 + 2, 0, False)

            @pl.when(t0 + 4 < T)
            def _():
                start_gather(t0 + 4, 0)

            unit(t0 + 3, 1, False)
            return c

        lax.fori_loop(0, (T - 2) // 2, pair, 0)
        wait_out(T - 2, 0)
        wait_out(T - 1, 1)

    return lookup


def kernel(x, table):
    S, T = x.shape
    V, D = table.shape
    xt = x.T
    t2 = _make_stage1(V, D)(table.T)
    out5 = _make_stage2(S, T, V, D)(xt, t2)
    return out5.transpose(2, 4, 0, 1, 3).reshape(S, T, D)
